# baseline (speedup 1.0000x reference)
import functools

import jax
import jax.numpy as jnp
from jax import lax
from jax.experimental import pallas as pl
from jax.experimental.pallas import tpu as pltpu
from jax.experimental.pallas import tpu_sc as plsc

N_NODES = 50000
N_EDGES = 800000
D_IN = 65
HID = 64
HID2 = 2 * HID

NC, NS, LANES = 2, 16, 16
NW = NC * NS
IDX_W = 128
CHUNK = 512
N_IDX = CHUNK // IDX_W
E_PAD = 802816
EW = E_PAD // NW
N_CH = EW // CHUNK

NHALF = N_NODES // 2
CHUNK_SC = 256
N_IDX_SC = CHUNK_SC // IDX_W
N_CH_SC = (E_PAD // NS) // CHUNK_SC
TRASH = 25088
ACC_ROWS = TRASH + 8
NZCOPY = TRASH // CHUNK_SC
STRIPE_O = 1563
LAST_O = NHALF - (NS - 1) * STRIPE_O

MLP_BR = 4096

_mesh = plsc.VectorSubcoreMesh(core_axis_name="c", subcore_axis_name="s")
_sc_params = pltpu.CompilerParams(use_tc_tiling_on_sc=False)


def _proj_body(x_ref, w_ref, o_ref):
    y = jnp.dot(x_ref[...], w_ref[...],
                preferred_element_type=jnp.float32).astype(jnp.bfloat16)
    o_ref[...] = jnp.concatenate([y, y], axis=-1)


def _proj(x, W1):
    return pl.pallas_call(
        _proj_body,
        grid=(25,),
        in_specs=[pl.BlockSpec((2000, D_IN), lambda i: (i, 0)),
                  pl.BlockSpec((D_IN, HID), lambda i: (0, 0))],
        out_specs=pl.BlockSpec((2000, HID2), lambda i: (i, 0)),
        out_shape=jax.ShapeDtypeStruct((N_NODES, HID2), jnp.bfloat16),
    )(x, W1)


@functools.partial(
    pl.kernel,
    mesh=_mesh,
    out_type=[jax.ShapeDtypeStruct((E_PAD, HID2), jnp.bfloat16),
              jax.ShapeDtypeStruct((E_PAD, HID2), jnp.bfloat16)],
    scratch_types=[pltpu.VMEM((N_IDX, IDX_W), jnp.int32),
                   pltpu.VMEM((N_IDX, IDX_W), jnp.int32),
                   pltpu.VMEM((CHUNK, HID2), jnp.bfloat16),
                   pltpu.VMEM((CHUNK, HID2), jnp.bfloat16),
                   pltpu.SemaphoreType.DMA,
                   pltpu.SemaphoreType.DMA],
    compiler_params=_sc_params,
)
def _gather_kernel(y_hbm, src_hbm, dst_hbm, gs_hbm, gd_hbm,
                   isv, idv, rs, rd, sem_i, sem_g):
    wid = lax.axis_index("s") * NC + lax.axis_index("c")

    @pl.loop(0, N_CH)
    def _(c):
        ch = wid * N_CH + c
        row0 = ch * N_IDX
        e0 = ch * CHUNK
        cp_s = pltpu.async_copy(src_hbm.at[pl.ds(row0, N_IDX)], isv, sem_i)
        cp_d = pltpu.async_copy(dst_hbm.at[pl.ds(row0, N_IDX)], idv, sem_i)
        cp_s.wait()
        cp_d.wait()
        cps = []
        for j in range(N_IDX):
            cps.append(pltpu.async_copy(
                y_hbm.at[isv.at[j]], rs.at[pl.ds(j * IDX_W, IDX_W)], sem_g))
            cps.append(pltpu.async_copy(
                y_hbm.at[idv.at[j]], rd.at[pl.ds(j * IDX_W, IDX_W)], sem_g))
        for cp in cps:
            cp.wait()
        pltpu.sync_copy(rs, gs_hbm.at[pl.ds(e0, CHUNK)])
        pltpu.sync_copy(rd, gd_hbm.at[pl.ds(e0, CHUNK)])


def _mlp_body(b1_ref, w2_ref, b2_ref, w3_ref, b3_ref, gs_ref, gd_ref, m_ref):
    i = pl.program_id(0)
    d = gs_ref[...].astype(jnp.float32) - gd_ref[...].astype(jnp.float32)
    h = jnp.maximum(d + b1_ref[...], 0.0)
    h = jnp.dot(h.astype(jnp.bfloat16), w2_ref[...],
                preferred_element_type=jnp.float32)
    h = jnp.maximum(h + b2_ref[...], 0.0)
    m = jnp.dot(h.astype(jnp.bfloat16), w3_ref[...],
                preferred_element_type=jnp.float32)
    m = m + b3_ref[...]
    row = i * MLP_BR + lax.broadcasted_iota(jnp.int32, m.shape, 0)
    m_ref[...] = jnp.where(row < N_EDGES, m, 0.0)


def _mlp(b1d, W2d, b2d, W3s, b3, gs, gd):
    grid = E_PAD // MLP_BR
    return pl.pallas_call(
        _mlp_body,
        grid=(grid,),
        in_specs=[pl.BlockSpec((1, HID2), lambda i: (0, 0)),
                  pl.BlockSpec((HID2, HID2), lambda i: (0, 0)),
                  pl.BlockSpec((1, HID2), lambda i: (0, 0)),
                  pl.BlockSpec((HID2, HID), lambda i: (0, 0)),
                  pl.BlockSpec((1, HID), lambda i: (0, 0)),
                  pl.BlockSpec((MLP_BR, HID2), lambda i: (i, 0)),
                  pl.BlockSpec((MLP_BR, HID2), lambda i: (i, 0))],
        out_specs=pl.BlockSpec((MLP_BR, HID), lambda i: (i, 0)),
        out_shape=jax.ShapeDtypeStruct((E_PAD, HID), jnp.float32),
    )(b1d, W2d, b2d, W3s, b3, gs, gd)


@functools.partial(
    pl.kernel,
    mesh=_mesh,
    out_type=jax.ShapeDtypeStruct((N_NODES, HID), jnp.float32),
    scratch_types=[pltpu.VMEM((N_IDX_SC, IDX_W), jnp.int32),
                   pltpu.VMEM((CHUNK_SC, HID), jnp.float32),
                   pltpu.VMEM_SHARED((ACC_ROWS, HID), jnp.float32),
                   pltpu.SemaphoreType.DMA],
    compiler_params=_sc_params,
)
def _scatter_kernel(m_hbm, dst_hbm, out_hbm, idxv, rows, acc, sem):
    cid = lax.axis_index("c")
    sid = lax.axis_index("s")

    @pl.loop(0, CHUNK_SC)
    def _(r):
        for q in range(HID // LANES):
            rows[r, pl.ds(q * LANES, LANES)] = jnp.zeros((LANES,),
                                                         jnp.float32)

    for k in range(7):
        zi = sid * 7 + k

        @pl.when(zi < NZCOPY)
        def _():
            pltpu.sync_copy(rows, acc.at[pl.ds(zi * CHUNK_SC, CHUNK_SC)])
    plsc.subcore_barrier()

    node0 = cid * NHALF

    @pl.loop(0, N_CH_SC)
    def _(c):
        ch = sid * N_CH_SC + c
        row0 = ch * N_IDX_SC
        e0 = ch * CHUNK_SC
        cp_i = pltpu.async_copy(dst_hbm.at[pl.ds(row0, N_IDX_SC)], idxv, sem)
        cp_m = pltpu.async_copy(m_hbm.at[pl.ds(e0, CHUNK_SC)], rows, sem)
        cp_i.wait()
        cp_m.wait()
        for j in range(N_IDX_SC):
            for q in range(IDX_W // LANES):
                v = idxv[j, pl.ds(q * LANES, LANES)]
                loc = v - node0
                ok = (loc >= 0) & (loc < NHALF)
                idxv[j, pl.ds(q * LANES, LANES)] = jnp.where(
                    ok, loc, TRASH + (v & 7))
        for j in range(N_IDX_SC):
            pltpu.sync_copy(rows.at[pl.ds(j * IDX_W, IDX_W)],
                            acc.at[idxv.at[j]], add=True)

    plsc.subcore_barrier()

    out0 = cid * NHALF

    @pl.when(sid < NS - 1)
    def _():
        pltpu.sync_copy(acc.at[pl.ds(sid * STRIPE_O, STRIPE_O)],
                        out_hbm.at[pl.ds(out0 + sid * STRIPE_O, STRIPE_O)])

    @pl.when(sid == NS - 1)
    def _():
        pltpu.sync_copy(acc.at[pl.ds((NS - 1) * STRIPE_O, LAST_O)],
                        out_hbm.at[pl.ds(out0 + (NS - 1) * STRIPE_O, LAST_O)])


def kernel(x, edge_index, W1, b1, W2, b2, W3, b3):
    src = edge_index[0]
    dst = edge_index[1]
    pad = E_PAD - N_EDGES
    src2d = jnp.concatenate(
        [src, jnp.zeros((pad,), jnp.int32)]).reshape(E_PAD // IDX_W, IDX_W)
    dst2d = jnp.concatenate(
        [dst, jnp.zeros((pad,), jnp.int32)]).reshape(E_PAD // IDX_W, IDX_W)

    zz = jnp.zeros((HID, HID), jnp.bfloat16)
    W2b = W2.astype(jnp.bfloat16)
    W2d = jnp.block([[W2b, zz], [zz, W2b]])
    W3s = jnp.concatenate([W3.astype(jnp.bfloat16),
                           jnp.zeros((HID, HID), jnp.bfloat16)], axis=0)
    b1d = jnp.tile(b1, 2).reshape(1, HID2)
    b2d = jnp.tile(b2, 2).reshape(1, HID2)

    y2 = _proj(x, W1)
    gs, gd = _gather_kernel(y2, src2d, dst2d)
    m = _mlp(b1d, W2d, b2d, W3s, b3.reshape(1, HID), gs, gd)
    return _scatter_kernel(m, dst2d)


# speedup vs baseline: 1.4051x; 1.4051x over previous
import functools

import jax
import jax.numpy as jnp
from jax import lax
from jax.experimental import pallas as pl
from jax.experimental.pallas import tpu as pltpu
from jax.experimental.pallas import tpu_sc as plsc

N_NODES = 50000
N_EDGES = 800000
D_IN = 65
HID = 64
HID2 = 2 * HID

NC, NS, LANES = 2, 16, 16
NW = NC * NS
IDX_W = 128
CHUNK = 256
N_IDX = CHUNK // IDX_W
E_PAD = 802816
EW = E_PAD // NW
N_CH = EW // CHUNK

NHALF = N_NODES // 2
CHUNK_SC = 256
N_IDX_SC = CHUNK_SC // IDX_W
N_CH_SC = (E_PAD // NS) // CHUNK_SC
TRASH = 25088
ACC_ROWS = TRASH + 8
NZCOPY = TRASH // CHUNK_SC
STRIPE_O = 1563
LAST_O = NHALF - (NS - 1) * STRIPE_O

MLP_BR = 2048

_mesh = plsc.VectorSubcoreMesh(core_axis_name="c", subcore_axis_name="s")
_sc_params = pltpu.CompilerParams(use_tc_tiling_on_sc=False)


def _proj_body(x_ref, w_ref, o_ref):
    y = jnp.dot(x_ref[...], w_ref[...], preferred_element_type=jnp.float32)
    o_ref[...] = jnp.concatenate([y, -y], axis=-1)


def _proj(x, W1):
    return pl.pallas_call(
        _proj_body,
        grid=(25,),
        in_specs=[pl.BlockSpec((2000, D_IN), lambda i: (i, 0)),
                  pl.BlockSpec((D_IN, HID), lambda i: (0, 0))],
        out_specs=pl.BlockSpec((2000, HID2), lambda i: (i, 0)),
        out_shape=jax.ShapeDtypeStruct((N_NODES, HID2), jnp.float32),
    )(x, W1)


@functools.partial(
    pl.kernel,
    mesh=_mesh,
    out_type=[jax.ShapeDtypeStruct((E_PAD, HID2), jnp.float32),
              jax.ShapeDtypeStruct((E_PAD, HID2), jnp.float32)],
    scratch_types=[pltpu.VMEM((N_IDX, IDX_W), jnp.int32),
                   pltpu.VMEM((N_IDX, IDX_W), jnp.int32),
                   pltpu.VMEM((CHUNK, HID2), jnp.float32),
                   pltpu.VMEM((CHUNK, HID2), jnp.float32),
                   pltpu.SemaphoreType.DMA,
                   pltpu.SemaphoreType.DMA],
    compiler_params=_sc_params,
)
def _gather_kernel(y_hbm, src_hbm, dst_hbm, gs_hbm, gd_hbm,
                   isv, idv, rs, rd, sem_i, sem_g):
    wid = lax.axis_index("s") * NC + lax.axis_index("c")

    @pl.loop(0, N_CH)
    def _(c):
        ch = wid * N_CH + c
        row0 = ch * N_IDX
        e0 = ch * CHUNK
        cp_s = pltpu.async_copy(src_hbm.at[pl.ds(row0, N_IDX)], isv, sem_i)
        cp_d = pltpu.async_copy(dst_hbm.at[pl.ds(row0, N_IDX)], idv, sem_i)
        cp_s.wait()
        cp_d.wait()
        cps = []
        for j in range(N_IDX):
            cps.append(pltpu.async_copy(
                y_hbm.at[isv.at[j]], rs.at[pl.ds(j * IDX_W, IDX_W)], sem_g))
            cps.append(pltpu.async_copy(
                y_hbm.at[idv.at[j]], rd.at[pl.ds(j * IDX_W, IDX_W)], sem_g))
        for cp in cps:
            cp.wait()
        pltpu.sync_copy(rs, gs_hbm.at[pl.ds(e0, CHUNK)])
        pltpu.sync_copy(rd, gd_hbm.at[pl.ds(e0, CHUNK)])


def _mlp_body(b1_ref, w2_ref, b2_ref, w3_ref, b3_ref, gs_ref, gd_ref, m_ref):
    i = pl.program_id(0)
    d = gs_ref[...] - gd_ref[...]
    h = jnp.maximum(d + b1_ref[...], 0.0)
    h = jnp.dot(h.astype(jnp.bfloat16), w2_ref[...],
                preferred_element_type=jnp.float32)
    h = jnp.maximum(h + b2_ref[...], 0.0)
    m = jnp.dot(h.astype(jnp.bfloat16), w3_ref[...],
                preferred_element_type=jnp.float32)
    m = m + b3_ref[...]
    row = i * MLP_BR + lax.broadcasted_iota(jnp.int32, m.shape, 0)
    m_ref[...] = jnp.where(row < N_EDGES, m, 0.0)


def _mlp(b1d, W2z, b2, W3, b3, gs, gd):
    grid = E_PAD // MLP_BR
    return pl.pallas_call(
        _mlp_body,
        grid=(grid,),
        in_specs=[pl.BlockSpec((1, HID2), lambda i: (0, 0)),
                  pl.BlockSpec((HID2, HID), lambda i: (0, 0)),
                  pl.BlockSpec((1, HID), lambda i: (0, 0)),
                  pl.BlockSpec((HID, HID), lambda i: (0, 0)),
                  pl.BlockSpec((1, HID), lambda i: (0, 0)),
                  pl.BlockSpec((MLP_BR, HID2), lambda i: (i, 0)),
                  pl.BlockSpec((MLP_BR, HID2), lambda i: (i, 0))],
        out_specs=pl.BlockSpec((MLP_BR, HID), lambda i: (i, 0)),
        out_shape=jax.ShapeDtypeStruct((E_PAD, HID), jnp.float32),
    )(b1d, W2z, b2, W3, b3, gs, gd)


@functools.partial(
    pl.kernel,
    mesh=_mesh,
    out_type=jax.ShapeDtypeStruct((N_NODES, HID), jnp.float32),
    scratch_types=[pltpu.VMEM((N_IDX_SC, IDX_W), jnp.int32),
                   pltpu.VMEM((CHUNK_SC, HID), jnp.float32),
                   pltpu.VMEM_SHARED((ACC_ROWS, HID), jnp.float32),
                   pltpu.SemaphoreType.DMA],
    compiler_params=_sc_params,
)
def _scatter_kernel(m_hbm, dst_hbm, out_hbm, idxv, rows, acc, sem):
    cid = lax.axis_index("c")
    sid = lax.axis_index("s")

    @pl.loop(0, CHUNK_SC)
    def _(r):
        for q in range(HID // LANES):
            rows[r, pl.ds(q * LANES, LANES)] = jnp.zeros((LANES,),
                                                         jnp.float32)

    for k in range(7):
        zi = sid * 7 + k

        @pl.when(zi < NZCOPY)
        def _():
            pltpu.sync_copy(rows, acc.at[pl.ds(zi * CHUNK_SC, CHUNK_SC)])
    plsc.subcore_barrier()

    node0 = cid * NHALF

    @pl.loop(0, N_CH_SC)
    def _(c):
        ch = sid * N_CH_SC + c
        row0 = ch * N_IDX_SC
        e0 = ch * CHUNK_SC
        cp_i = pltpu.async_copy(dst_hbm.at[pl.ds(row0, N_IDX_SC)], idxv, sem)
        cp_m = pltpu.async_copy(m_hbm.at[pl.ds(e0, CHUNK_SC)], rows, sem)
        cp_i.wait()
        cp_m.wait()
        for j in range(N_IDX_SC):
            for q in range(IDX_W // LANES):
                v = idxv[j, pl.ds(q * LANES, LANES)]
                loc = v - node0
                ok = (loc >= 0) & (loc < NHALF)
                idxv[j, pl.ds(q * LANES, LANES)] = jnp.where(
                    ok, loc, TRASH + (v & 7))
        for j in range(N_IDX_SC):
            pltpu.sync_copy(rows.at[pl.ds(j * IDX_W, IDX_W)],
                            acc.at[idxv.at[j]], add=True)

    plsc.subcore_barrier()

    out0 = cid * NHALF

    @pl.when(sid < NS - 1)
    def _():
        pltpu.sync_copy(acc.at[pl.ds(sid * STRIPE_O, STRIPE_O)],
                        out_hbm.at[pl.ds(out0 + sid * STRIPE_O, STRIPE_O)])

    @pl.when(sid == NS - 1)
    def _():
        pltpu.sync_copy(acc.at[pl.ds((NS - 1) * STRIPE_O, LAST_O)],
                        out_hbm.at[pl.ds(out0 + (NS - 1) * STRIPE_O, LAST_O)])


def kernel(x, edge_index, W1, b1, W2, b2, W3, b3):
    src = edge_index[0]
    dst = edge_index[1]
    pad = E_PAD - N_EDGES
    src2d = jnp.concatenate(
        [src, jnp.zeros((pad,), jnp.int32)]).reshape(E_PAD // IDX_W, IDX_W)
    dst2d = jnp.concatenate(
        [dst, jnp.zeros((pad,), jnp.int32)]).reshape(E_PAD // IDX_W, IDX_W)

    W2z = jnp.concatenate([W2.astype(jnp.bfloat16),
                           jnp.zeros((HID, HID), jnp.bfloat16)], axis=0)
    b1d = jnp.concatenate([b1, jnp.zeros((HID,), jnp.float32)]).reshape(
        1, HID2)

    y2 = _proj(x, W1)
    gs, gd = _gather_kernel(y2, src2d, dst2d)
    m = _mlp(b1d, W2z, b2.reshape(1, HID), W3.astype(jnp.bfloat16),
             b3.reshape(1, HID), gs, gd)
    return _scatter_kernel(m, dst2d)


# speedup vs baseline: 1.4691x; 1.0456x over previous
import functools

import jax
import jax.numpy as jnp
from jax import lax
from jax.experimental import pallas as pl
from jax.experimental.pallas import tpu as pltpu
from jax.experimental.pallas import tpu_sc as plsc

N_NODES = 50000
N_EDGES = 800000
D_IN = 65
HID = 64
HID2 = 2 * HID

NC, NS, LANES = 2, 16, 16
NW = NC * NS
IDX_W = 128
CHUNK = 128
E_PAD = 802816
EW = E_PAD // NW
N_CH = EW // CHUNK

NHALF = N_NODES // 2
CHUNK_SC = 256
N_IDX_SC = CHUNK_SC // IDX_W
N_CH_SC = (E_PAD // NS) // CHUNK_SC
TRASH = 25088
ACC_ROWS = TRASH + 8
NZCOPY = TRASH // CHUNK_SC
STRIPE_O = 1563
LAST_O = NHALF - (NS - 1) * STRIPE_O

MLP_BR = 2048

_mesh = plsc.VectorSubcoreMesh(core_axis_name="c", subcore_axis_name="s")
_sc_params = pltpu.CompilerParams(use_tc_tiling_on_sc=False)


def _proj_body(x_ref, w_ref, o_ref):
    y = jnp.dot(x_ref[...], w_ref[...], preferred_element_type=jnp.float32)
    o_ref[...] = jnp.concatenate([y, -y], axis=-1)


def _proj(x, W1):
    return pl.pallas_call(
        _proj_body,
        grid=(25,),
        in_specs=[pl.BlockSpec((2000, D_IN), lambda i: (i, 0)),
                  pl.BlockSpec((D_IN, HID), lambda i: (0, 0))],
        out_specs=pl.BlockSpec((2000, HID2), lambda i: (i, 0)),
        out_shape=jax.ShapeDtypeStruct((N_NODES, HID2), jnp.float32),
    )(x, W1)


@functools.partial(
    pl.kernel,
    mesh=_mesh,
    out_type=[jax.ShapeDtypeStruct((E_PAD, HID2), jnp.float32),
              jax.ShapeDtypeStruct((E_PAD, HID2), jnp.float32)],
    scratch_types=[pltpu.VMEM((N_CH, IDX_W), jnp.int32),
                   pltpu.VMEM((N_CH, IDX_W), jnp.int32),
                   pltpu.VMEM((CHUNK, HID2), jnp.float32),
                   pltpu.VMEM((CHUNK, HID2), jnp.float32),
                   pltpu.VMEM((CHUNK, HID2), jnp.float32),
                   pltpu.VMEM((CHUNK, HID2), jnp.float32),
                   pltpu.SemaphoreType.DMA,
                   pltpu.SemaphoreType.DMA,
                   pltpu.SemaphoreType.DMA],
    compiler_params=_sc_params,
)
def _gather_kernel(y_hbm, src_hbm, dst_hbm, gs_hbm, gd_hbm,
                   isv, idv, rs0, rd0, rs1, rd1, sem_g, sem_w0, sem_w1):
    wid = lax.axis_index("s") * NC + lax.axis_index("c")
    cp_s = pltpu.async_copy(src_hbm.at[pl.ds(wid * N_CH, N_CH)], isv, sem_g)
    cp_d = pltpu.async_copy(dst_hbm.at[pl.ds(wid * N_CH, N_CH)], idv, sem_g)
    cp_s.wait()
    cp_d.wait()
    base_e = wid * EW

    @pl.loop(0, N_CH // 2)
    def _(p):
        for x, (rs, rd, sem_w) in enumerate(((rs0, rd0, sem_w0),
                                             (rs1, rd1, sem_w1))):
            ch = 2 * p + x

            @pl.when(p > 0)
            def _():
                pltpu.make_async_copy(
                    rs, gs_hbm.at[pl.ds(0, CHUNK)], sem_w).wait()
                pltpu.make_async_copy(
                    rd, gd_hbm.at[pl.ds(0, CHUNK)], sem_w).wait()

            cg1 = pltpu.async_copy(y_hbm.at[isv.at[ch]], rs, sem_g)
            cg2 = pltpu.async_copy(y_hbm.at[idv.at[ch]], rd, sem_g)
            cg1.wait()
            cg2.wait()
            e0 = base_e + ch * CHUNK
            pltpu.async_copy(rs, gs_hbm.at[pl.ds(e0, CHUNK)], sem_w)
            pltpu.async_copy(rd, gd_hbm.at[pl.ds(e0, CHUNK)], sem_w)

    for rs, rd, sem_w in ((rs0, rd0, sem_w0), (rs1, rd1, sem_w1)):
        pltpu.make_async_copy(rs, gs_hbm.at[pl.ds(0, CHUNK)], sem_w).wait()
        pltpu.make_async_copy(rd, gd_hbm.at[pl.ds(0, CHUNK)], sem_w).wait()


def _mlp_body(b1_ref, w2_ref, b2_ref, w3_ref, b3_ref, gs_ref, gd_ref, m_ref):
    i = pl.program_id(0)
    d = gs_ref[...] - gd_ref[...]
    h = jnp.maximum(d + b1_ref[...], 0.0)
    h = jnp.dot(h.astype(jnp.bfloat16), w2_ref[...],
                preferred_element_type=jnp.float32)
    h = jnp.maximum(h + b2_ref[...], 0.0)
    m = jnp.dot(h.astype(jnp.bfloat16), w3_ref[...],
                preferred_element_type=jnp.float32)
    m = m + b3_ref[...]
    row = i * MLP_BR + lax.broadcasted_iota(jnp.int32, m.shape, 0)
    m_ref[...] = jnp.where(row < N_EDGES, m, 0.0)


def _mlp(b1d, W2z, b2, W3, b3, gs, gd):
    grid = E_PAD // MLP_BR
    return pl.pallas_call(
        _mlp_body,
        grid=(grid,),
        in_specs=[pl.BlockSpec((1, HID2), lambda i: (0, 0)),
                  pl.BlockSpec((HID2, HID), lambda i: (0, 0)),
                  pl.BlockSpec((1, HID), lambda i: (0, 0)),
                  pl.BlockSpec((HID, HID), lambda i: (0, 0)),
                  pl.BlockSpec((1, HID), lambda i: (0, 0)),
                  pl.BlockSpec((MLP_BR, HID2), lambda i: (i, 0)),
                  pl.BlockSpec((MLP_BR, HID2), lambda i: (i, 0))],
        out_specs=pl.BlockSpec((MLP_BR, HID), lambda i: (i, 0)),
        out_shape=jax.ShapeDtypeStruct((E_PAD, HID), jnp.float32),
    )(b1d, W2z, b2, W3, b3, gs, gd)


@functools.partial(
    pl.kernel,
    mesh=_mesh,
    out_type=jax.ShapeDtypeStruct((N_NODES, HID), jnp.float32),
    scratch_types=[pltpu.VMEM((N_IDX_SC, IDX_W), jnp.int32),
                   pltpu.VMEM((CHUNK_SC, HID), jnp.float32),
                   pltpu.VMEM_SHARED((ACC_ROWS, HID), jnp.float32),
                   pltpu.SemaphoreType.DMA],
    compiler_params=_sc_params,
)
def _scatter_kernel(m_hbm, dst_hbm, out_hbm, idxv, rows, acc, sem):
    cid = lax.axis_index("c")
    sid = lax.axis_index("s")

    @pl.loop(0, CHUNK_SC)
    def _(r):
        for q in range(HID // LANES):
            rows[r, pl.ds(q * LANES, LANES)] = jnp.zeros((LANES,),
                                                         jnp.float32)

    for k in range(7):
        zi = sid * 7 + k

        @pl.when(zi < NZCOPY)
        def _():
            pltpu.sync_copy(rows, acc.at[pl.ds(zi * CHUNK_SC, CHUNK_SC)])
    plsc.subcore_barrier()

    node0 = cid * NHALF

    @pl.loop(0, N_CH_SC)
    def _(c):
        ch = sid * N_CH_SC + c
        row0 = ch * N_IDX_SC
        e0 = ch * CHUNK_SC
        cp_i = pltpu.async_copy(dst_hbm.at[pl.ds(row0, N_IDX_SC)], idxv, sem)
        cp_m = pltpu.async_copy(m_hbm.at[pl.ds(e0, CHUNK_SC)], rows, sem)
        cp_i.wait()
        cp_m.wait()
        for j in range(N_IDX_SC):
            for q in range(IDX_W // LANES):
                v = idxv[j, pl.ds(q * LANES, LANES)]
                loc = v - node0
                ok = (loc >= 0) & (loc < NHALF)
                idxv[j, pl.ds(q * LANES, LANES)] = jnp.where(
                    ok, loc, TRASH + (v & 7))
        for j in range(N_IDX_SC):
            pltpu.sync_copy(rows.at[pl.ds(j * IDX_W, IDX_W)],
                            acc.at[idxv.at[j]], add=True)

    plsc.subcore_barrier()

    out0 = cid * NHALF

    @pl.when(sid < NS - 1)
    def _():
        pltpu.sync_copy(acc.at[pl.ds(sid * STRIPE_O, STRIPE_O)],
                        out_hbm.at[pl.ds(out0 + sid * STRIPE_O, STRIPE_O)])

    @pl.when(sid == NS - 1)
    def _():
        pltpu.sync_copy(acc.at[pl.ds((NS - 1) * STRIPE_O, LAST_O)],
                        out_hbm.at[pl.ds(out0 + (NS - 1) * STRIPE_O, LAST_O)])


def kernel(x, edge_index, W1, b1, W2, b2, W3, b3):
    src = edge_index[0]
    dst = edge_index[1]
    pad = E_PAD - N_EDGES
    src2d = jnp.concatenate(
        [src, jnp.zeros((pad,), jnp.int32)]).reshape(E_PAD // IDX_W, IDX_W)
    dst2d = jnp.concatenate(
        [dst, jnp.zeros((pad,), jnp.int32)]).reshape(E_PAD // IDX_W, IDX_W)

    W2z = jnp.concatenate([W2.astype(jnp.bfloat16),
                           jnp.zeros((HID, HID), jnp.bfloat16)], axis=0)
    b1d = jnp.concatenate([b1, jnp.zeros((HID,), jnp.float32)]).reshape(
        1, HID2)

    y2 = _proj(x, W1)
    gs, gd = _gather_kernel(y2, src2d, dst2d)
    m = _mlp(b1d, W2z, b2.reshape(1, HID), W3.astype(jnp.bfloat16),
             b3.reshape(1, HID), gs, gd)
    return _scatter_kernel(m, dst2d)


# speedup vs baseline: 1.5499x; 1.0550x over previous
import functools

import jax
import jax.numpy as jnp
from jax import lax
from jax.experimental import pallas as pl
from jax.experimental.pallas import tpu as pltpu
from jax.experimental.pallas import tpu_sc as plsc

N_NODES = 50000
N_EDGES = 800000
D_IN = 65
HID = 64
HID2 = 2 * HID

NC, NS, LANES = 2, 16, 16
NW = NC * NS
IDX_W = 128
CHUNK = 128
E_PAD = 802816
K_SPLIT = 2
E_K = E_PAD // K_SPLIT
EW = E_K // NW
N_CH = EW // CHUNK

NHALF = N_NODES // 2
CHUNK_SC = 256
N_IDX_SC = CHUNK_SC // IDX_W
N_CH_SC = (E_K // NS) // CHUNK_SC
TRASH = 25088
ACC_ROWS = TRASH + 8
NZCOPY = TRASH // CHUNK_SC
STRIPE_O = 1563
LAST_O = NHALF - (NS - 1) * STRIPE_O

MLP_BR = 2048

_mesh = plsc.VectorSubcoreMesh(core_axis_name="c", subcore_axis_name="s")
_sc_params = pltpu.CompilerParams(use_tc_tiling_on_sc=False)


def _proj_body(x_ref, w_ref, o_ref):
    y = jnp.dot(x_ref[...], w_ref[...], preferred_element_type=jnp.float32)
    o_ref[...] = jnp.concatenate([y, -y], axis=-1)


def _proj(x, W1):
    return pl.pallas_call(
        _proj_body,
        grid=(25,),
        in_specs=[pl.BlockSpec((2000, D_IN), lambda i: (i, 0)),
                  pl.BlockSpec((D_IN, HID), lambda i: (0, 0))],
        out_specs=pl.BlockSpec((2000, HID2), lambda i: (i, 0)),
        out_shape=jax.ShapeDtypeStruct((N_NODES, HID2), jnp.float32),
    )(x, W1)


@functools.partial(
    pl.kernel,
    mesh=_mesh,
    out_type=[jax.ShapeDtypeStruct((E_K, HID2), jnp.float32),
              jax.ShapeDtypeStruct((E_K, HID2), jnp.float32)],
    scratch_types=[pltpu.VMEM((N_CH, IDX_W), jnp.int32),
                   pltpu.VMEM((N_CH, IDX_W), jnp.int32),
                   pltpu.VMEM((CHUNK, HID2), jnp.float32),
                   pltpu.VMEM((CHUNK, HID2), jnp.float32),
                   pltpu.VMEM((CHUNK, HID2), jnp.float32),
                   pltpu.VMEM((CHUNK, HID2), jnp.float32),
                   pltpu.SemaphoreType.DMA,
                   pltpu.SemaphoreType.DMA,
                   pltpu.SemaphoreType.DMA],
    compiler_params=_sc_params,
)
def _gather_kernel(y_hbm, src_hbm, dst_hbm, gs_hbm, gd_hbm,
                   isv, idv, rs0, rd0, rs1, rd1, sem_g, sem_w0, sem_w1):
    wid = lax.axis_index("s") * NC + lax.axis_index("c")
    cp_s = pltpu.async_copy(src_hbm.at[pl.ds(wid * N_CH, N_CH)], isv, sem_g)
    cp_d = pltpu.async_copy(dst_hbm.at[pl.ds(wid * N_CH, N_CH)], idv, sem_g)
    cp_s.wait()
    cp_d.wait()
    base_e = wid * EW

    @pl.loop(0, N_CH // 2)
    def _(p):
        for x, (rs, rd, sem_w) in enumerate(((rs0, rd0, sem_w0),
                                             (rs1, rd1, sem_w1))):
            ch = 2 * p + x

            @pl.when(p > 0)
            def _():
                pltpu.make_async_copy(
                    rs, gs_hbm.at[pl.ds(0, CHUNK)], sem_w).wait()
                pltpu.make_async_copy(
                    rd, gd_hbm.at[pl.ds(0, CHUNK)], sem_w).wait()

            cg1 = pltpu.async_copy(y_hbm.at[isv.at[ch]], rs, sem_g)
            cg2 = pltpu.async_copy(y_hbm.at[idv.at[ch]], rd, sem_g)
            cg1.wait()
            cg2.wait()
            e0 = base_e + ch * CHUNK
            pltpu.async_copy(rs, gs_hbm.at[pl.ds(e0, CHUNK)], sem_w)
            pltpu.async_copy(rd, gd_hbm.at[pl.ds(e0, CHUNK)], sem_w)

    for rs, rd, sem_w in ((rs0, rd0, sem_w0), (rs1, rd1, sem_w1)):
        pltpu.make_async_copy(rs, gs_hbm.at[pl.ds(0, CHUNK)], sem_w).wait()
        pltpu.make_async_copy(rd, gd_hbm.at[pl.ds(0, CHUNK)], sem_w).wait()


def _mlp_body(base, b1_ref, w2_ref, b2_ref, w3_ref, b3_ref,
              gs_ref, gd_ref, m_ref):
    i = pl.program_id(0)
    d = gs_ref[...] - gd_ref[...]
    h = jnp.maximum(d + b1_ref[...], 0.0)
    h = jnp.dot(h.astype(jnp.bfloat16), w2_ref[...],
                preferred_element_type=jnp.float32)
    h = jnp.maximum(h + b2_ref[...], 0.0)
    m = jnp.dot(h.astype(jnp.bfloat16), w3_ref[...],
                preferred_element_type=jnp.float32)
    m = m + b3_ref[...]
    row = base + i * MLP_BR + lax.broadcasted_iota(jnp.int32, m.shape, 0)
    m_ref[...] = jnp.where(row < N_EDGES, m, 0.0)


def _mlp(base, b1d, W2z, b2, W3, b3, gs, gd):
    grid = E_K // MLP_BR
    return pl.pallas_call(
        functools.partial(_mlp_body, base),
        grid=(grid,),
        in_specs=[pl.BlockSpec((1, HID2), lambda i: (0, 0)),
                  pl.BlockSpec((HID2, HID), lambda i: (0, 0)),
                  pl.BlockSpec((1, HID), lambda i: (0, 0)),
                  pl.BlockSpec((HID, HID), lambda i: (0, 0)),
                  pl.BlockSpec((1, HID), lambda i: (0, 0)),
                  pl.BlockSpec((MLP_BR, HID2), lambda i: (i, 0)),
                  pl.BlockSpec((MLP_BR, HID2), lambda i: (i, 0))],
        out_specs=pl.BlockSpec((MLP_BR, HID), lambda i: (i, 0)),
        out_shape=jax.ShapeDtypeStruct((E_K, HID), jnp.float32),
    )(b1d, W2z, b2, W3, b3, gs, gd)


@functools.partial(
    pl.kernel,
    mesh=_mesh,
    out_type=jax.ShapeDtypeStruct((N_NODES, HID), jnp.float32),
    scratch_types=[pltpu.VMEM((N_IDX_SC, IDX_W), jnp.int32),
                   pltpu.VMEM((CHUNK_SC, HID), jnp.float32),
                   pltpu.VMEM_SHARED((ACC_ROWS, HID), jnp.float32),
                   pltpu.SemaphoreType.DMA],
    compiler_params=_sc_params,
)
def _scatter_kernel(m0_hbm, m1_hbm, dst_hbm, out_hbm, idxv, rows, acc, sem):
    cid = lax.axis_index("c")
    sid = lax.axis_index("s")

    @pl.loop(0, CHUNK_SC)
    def _(r):
        for q in range(HID // LANES):
            rows[r, pl.ds(q * LANES, LANES)] = jnp.zeros((LANES,),
                                                         jnp.float32)

    for k in range(7):
        zi = sid * 7 + k

        @pl.when(zi < NZCOPY)
        def _():
            pltpu.sync_copy(rows, acc.at[pl.ds(zi * CHUNK_SC, CHUNK_SC)])
    plsc.subcore_barrier()

    node0 = cid * NHALF

    for k, m_hbm in enumerate((m0_hbm, m1_hbm)):

        @pl.loop(0, N_CH_SC)
        def _(c):
            ch = sid * N_CH_SC + c
            row0 = k * (E_K // IDX_W) + ch * N_IDX_SC
            e0 = ch * CHUNK_SC
            cp_i = pltpu.async_copy(dst_hbm.at[pl.ds(row0, N_IDX_SC)],
                                    idxv, sem)
            cp_m = pltpu.async_copy(m_hbm.at[pl.ds(e0, CHUNK_SC)], rows, sem)
            cp_i.wait()
            cp_m.wait()
            for j in range(N_IDX_SC):
                for q in range(IDX_W // LANES):
                    v = idxv[j, pl.ds(q * LANES, LANES)]
                    loc = v - node0
                    ok = (loc >= 0) & (loc < NHALF)
                    idxv[j, pl.ds(q * LANES, LANES)] = jnp.where(
                        ok, loc, TRASH + (v & 7))
            for j in range(N_IDX_SC):
                pltpu.sync_copy(rows.at[pl.ds(j * IDX_W, IDX_W)],
                                acc.at[idxv.at[j]], add=True)

    plsc.subcore_barrier()

    out0 = cid * NHALF

    @pl.when(sid < NS - 1)
    def _():
        pltpu.sync_copy(acc.at[pl.ds(sid * STRIPE_O, STRIPE_O)],
                        out_hbm.at[pl.ds(out0 + sid * STRIPE_O, STRIPE_O)])

    @pl.when(sid == NS - 1)
    def _():
        pltpu.sync_copy(acc.at[pl.ds((NS - 1) * STRIPE_O, LAST_O)],
                        out_hbm.at[pl.ds(out0 + (NS - 1) * STRIPE_O, LAST_O)])


def kernel(x, edge_index, W1, b1, W2, b2, W3, b3):
    src = edge_index[0]
    dst = edge_index[1]
    pad = E_PAD - N_EDGES
    src2d = jnp.concatenate(
        [src, jnp.zeros((pad,), jnp.int32)]).reshape(E_PAD // IDX_W, IDX_W)
    dst2d = jnp.concatenate(
        [dst, jnp.zeros((pad,), jnp.int32)]).reshape(E_PAD // IDX_W, IDX_W)

    W2z = jnp.concatenate([W2.astype(jnp.bfloat16),
                           jnp.zeros((HID, HID), jnp.bfloat16)], axis=0)
    b1d = jnp.concatenate([b1, jnp.zeros((HID,), jnp.float32)]).reshape(
        1, HID2)

    y2 = _proj(x, W1)
    rows_k = E_K // IDX_W
    ms = []
    for k in range(K_SPLIT):
        s_k = lax.slice_in_dim(src2d, k * rows_k, (k + 1) * rows_k, axis=0)
        d_k = lax.slice_in_dim(dst2d, k * rows_k, (k + 1) * rows_k, axis=0)
        gs, gd = _gather_kernel(y2, s_k, d_k)
        ms.append(_mlp(k * E_K, b1d, W2z, b2.reshape(1, HID),
                       W3.astype(jnp.bfloat16), b3.reshape(1, HID), gs, gd))
    return _scatter_kernel(ms[0], ms[1], dst2d)
